# baseline (speedup 1.0000x reference)
import functools

import jax
import jax.numpy as jnp
from jax.experimental import pallas as pl

N = 10000
D = 128
NEG_SLOPE = 0.2


def _mm_kernel(x_ref, w_ref, b_ref, o_ref, *, activate):
    acc = jnp.dot(x_ref[...], w_ref[...], preferred_element_type=jnp.float32)
    acc = acc + b_ref[...][None, :]
    if activate:
        acc = jnp.maximum(acc, 0.0)
    o_ref[...] = acc


def _dense(x, w, b, activate):
    n, d = x.shape
    dout = w.shape[1]
    blk = 400
    return pl.pallas_call(
        functools.partial(_mm_kernel, activate=activate),
        grid=(n // blk,),
        in_specs=[
            pl.BlockSpec((blk, d), lambda i: (i, 0)),
            pl.BlockSpec((d, dout), lambda i: (0, 0)),
            pl.BlockSpec((dout,), lambda i: (0,)),
        ],
        out_specs=pl.BlockSpec((blk, dout), lambda i: (i, 0)),
        out_shape=jax.ShapeDtypeStruct((n, dout), jnp.float32),
    )(x, w, b)


def _segment_softmax(logits, seg, num_segments):
    maxs = jax.ops.segment_max(logits, seg, num_segments)
    maxs = jnp.where(jnp.isfinite(maxs), maxs, jnp.zeros_like(maxs))
    ex = jnp.exp(logits - maxs[seg])
    sums = jax.ops.segment_sum(ex, seg, num_segments)
    return ex / sums[seg]


def kernel(nodes, senders, receivers, enc_w, enc_b, attn0_w, attn0_b, core0_w, core0_b, attn1_w, attn1_b, core1_w, core1_b, dec_w, dec_b):
    x = _dense(nodes, enc_w, enc_b, True)
    hops = ((attn0_w, attn0_b, core0_w, core0_b), (attn1_w, attn1_b, core1_w, core1_b))
    for (aw_, ab_, cw_, cb_) in hops:
        s_idx, r_idx = receivers, senders
        a_src = x @ aw_[:D]
        a_dst = x @ aw_[D:]
        raw = a_src[s_idx] + a_dst[r_idx] + ab_
        nl = jax.nn.leaky_relu(raw, NEG_SLOPE)
        attn = _segment_softmax(nl[:, 0], r_idx, N)[:, None]
        weighted_edges = attn * x[s_idx]
        agg = jax.ops.segment_sum(weighted_edges, r_idx, N)
        x = _dense(agg, cw_, cb_, True) + agg
    out = _dense(x, dec_w, dec_b, False)
    return out


# speedup vs baseline: 11.4452x; 11.4452x over previous
import functools

import jax
import jax.numpy as jnp
from jax import lax
from jax.experimental import pallas as pl
from jax.experimental.pallas import tpu as pltpu
from jax.experimental.pallas import tpu_sc as plsc

N = 10000
E = 320000
D = 128
C = 40
NEG_SLOPE = 0.2

NC = 2
NS = 16
NW = NC * NS
N_PAD = 10240
EPT = E // NW + 240
assert EPT == 10240
K = 128
NCHUNK = EPT // K
STRIPE = N_PAD // NS


def _tc1_body(nodes_ref, ew_ref, eb_ref, aw_ref, ab_ref, x_ref, sc_ref):
    x = jnp.maximum(jnp.dot(nodes_ref[...], ew_ref[...],
                            preferred_element_type=jnp.float32)
                    + eb_ref[...][None, :], 0.0)
    x_ref[...] = x
    a_s = jnp.dot(x, aw_ref[...][:D, :], preferred_element_type=jnp.float32)
    a_d = jnp.dot(x, aw_ref[...][D:, :], preferred_element_type=jnp.float32)
    a_d = a_d + ab_ref[...][None, :]
    sc_ref[...] = jnp.concatenate([a_s, a_d], axis=1)


def _tc2_body(agg_ref, cw_ref, cb_ref, aw_ref, ab_ref, x_ref, sc_ref):
    agg = agg_ref[0] + agg_ref[1]
    x = jnp.maximum(jnp.dot(agg, cw_ref[...],
                            preferred_element_type=jnp.float32)
                    + cb_ref[...][None, :], 0.0) + agg
    x_ref[...] = x
    a_s = jnp.dot(x, aw_ref[...][:D, :], preferred_element_type=jnp.float32)
    a_d = jnp.dot(x, aw_ref[...][D:, :], preferred_element_type=jnp.float32)
    a_d = a_d + ab_ref[...][None, :]
    sc_ref[...] = jnp.concatenate([a_s, a_d], axis=1)


def _tc3_body(agg_ref, cw_ref, cb_ref, dw_ref, db_ref, out_ref):
    agg = agg_ref[0] + agg_ref[1]
    x = jnp.maximum(jnp.dot(agg, cw_ref[...],
                            preferred_element_type=jnp.float32)
                    + cb_ref[...][None, :], 0.0) + agg
    out_ref[...] = jnp.dot(x, dw_ref[...],
                           preferred_element_type=jnp.float32) + db_ref[...][None, :]


_BLK = 1024


def _tc_encode(nodes_p, enc_w, enc_b, aw, ab):
    return pl.pallas_call(
        _tc1_body,
        grid=(N_PAD // _BLK,),
        in_specs=[
            pl.BlockSpec((_BLK, D), lambda i: (i, 0)),
            pl.BlockSpec((D, D), lambda i: (0, 0)),
            pl.BlockSpec((D,), lambda i: (0,)),
            pl.BlockSpec((2 * D, 1), lambda i: (0, 0)),
            pl.BlockSpec((1,), lambda i: (0,)),
        ],
        out_specs=[
            pl.BlockSpec((_BLK, D), lambda i: (i, 0)),
            pl.BlockSpec((_BLK, 2), lambda i: (i, 0)),
        ],
        out_shape=[
            jax.ShapeDtypeStruct((N_PAD, D), jnp.float32),
            jax.ShapeDtypeStruct((N_PAD, 2), jnp.float32),
        ],
    )(nodes_p, enc_w, enc_b, aw, ab)


def _tc_update(agg2, cw, cb, aw, ab):
    return pl.pallas_call(
        _tc2_body,
        grid=(N_PAD // _BLK,),
        in_specs=[
            pl.BlockSpec((2, _BLK, D), lambda i: (0, i, 0)),
            pl.BlockSpec((D, D), lambda i: (0, 0)),
            pl.BlockSpec((D,), lambda i: (0,)),
            pl.BlockSpec((2 * D, 1), lambda i: (0, 0)),
            pl.BlockSpec((1,), lambda i: (0,)),
        ],
        out_specs=[
            pl.BlockSpec((_BLK, D), lambda i: (i, 0)),
            pl.BlockSpec((_BLK, 2), lambda i: (i, 0)),
        ],
        out_shape=[
            jax.ShapeDtypeStruct((N_PAD, D), jnp.float32),
            jax.ShapeDtypeStruct((N_PAD, 2), jnp.float32),
        ],
    )(agg2, cw, cb, aw, ab)


def _tc_decode(agg2, cw, cb, dw, db):
    return pl.pallas_call(
        _tc3_body,
        grid=(N_PAD // _BLK,),
        in_specs=[
            pl.BlockSpec((2, _BLK, D), lambda i: (0, i, 0)),
            pl.BlockSpec((D, D), lambda i: (0, 0)),
            pl.BlockSpec((D,), lambda i: (0,)),
            pl.BlockSpec((D, C), lambda i: (0, 0)),
            pl.BlockSpec((C,), lambda i: (0,)),
        ],
        out_specs=pl.BlockSpec((_BLK, C), lambda i: (i, 0)),
        out_shape=jax.ShapeDtypeStruct((N_PAD, C), jnp.float32),
    )(agg2, cw, cb, dw, db)


@functools.cache
def _make_sc_pass_a():
  mesh = plsc.VectorSubcoreMesh(core_axis_name="c", subcore_axis_name="s")

  @functools.partial(
      pl.kernel,
      mesh=mesh,
      out_type=[
          jax.ShapeDtypeStruct((NW, EPT), jnp.float32),
          jax.ShapeDtypeStruct((NC, N_PAD), jnp.float32),
      ],
      compiler_params=pltpu.CompilerParams(needs_layout_passes=False),
      scratch_types=[
          pltpu.VMEM((2 * N_PAD,), jnp.float32),
          pltpu.VMEM((EPT,), jnp.int32),
          pltpu.VMEM((EPT,), jnp.int32),
          pltpu.VMEM((NCHUNK, K), jnp.int32),
          pltpu.VMEM((EPT,), jnp.float32),
          pltpu.VMEM((STRIPE,), jnp.float32),
          pltpu.VMEM_SHARED((N_PAD,), jnp.float32),
      ],
  )
  def _sc_pass_a(scores_hbm, sidx_hbm, ridx_hbm, ridx2_hbm, ex_hbm, denom_hbm,
                 scores_v, sidx_v, ridxv_v, ridx2_v, ex_v, zb_v, denom_sp):
    cid = lax.axis_index("c")
    sid = lax.axis_index("s")
    wid = sid * NC + cid

    pltpu.sync_copy(scores_hbm, scores_v)
    pltpu.sync_copy(sidx_hbm.at[wid], sidx_v)
    pltpu.sync_copy(ridx_hbm.at[wid], ridxv_v)
    pltpu.sync_copy(ridx2_hbm.at[wid], ridx2_v)

    def _zero(i, carry):
        zb_v[pl.ds(i * 16, 16)] = jnp.zeros((16,), jnp.float32)
        return carry
    lax.fori_loop(0, STRIPE // 16, _zero, 0)
    pltpu.sync_copy(zb_v, denom_sp.at[pl.ds(sid * STRIPE, STRIPE)])
    plsc.subcore_barrier()

    def _chunk(cc, carry):
        base = cc * K
        for v in range(K // 16):
            off = base + v * 16
            si = sidx_v[pl.ds(off, 16)]
            ri = ridxv_v[pl.ds(off, 16)]
            a_s = plsc.load_gather(scores_v, [si * 2])
            a_d = plsc.load_gather(scores_v, [ri * 2 + 1])
            t = a_s + a_d
            nl = jnp.maximum(t, t * NEG_SLOPE)
            ex_v[pl.ds(off, 16)] = jnp.exp(nl)
        pltpu.sync_copy(ex_v.at[pl.ds(base, K)],
                        denom_sp.at[ridx2_v.at[cc]], add=True)
        return carry
    lax.fori_loop(0, NCHUNK, _chunk, 0)

    pltpu.sync_copy(ex_v, ex_hbm.at[wid])
    plsc.subcore_barrier()
    pltpu.sync_copy(denom_sp.at[pl.ds(sid * STRIPE, STRIPE)], zb_v)
    pltpu.sync_copy(zb_v, denom_hbm.at[cid].at[pl.ds(sid * STRIPE, STRIPE)])

  return _sc_pass_a


@functools.cache
def _make_sc_pass_c():
  mesh = plsc.VectorSubcoreMesh(core_axis_name="c", subcore_axis_name="s")

  @functools.partial(
      pl.kernel,
      mesh=mesh,
      out_type=[
          jax.ShapeDtypeStruct((NC, N_PAD, D), jnp.float32),
      ],
      compiler_params=pltpu.CompilerParams(needs_layout_passes=False),
      scratch_types=[
          pltpu.VMEM((K,), jnp.int32),
          pltpu.VMEM((K,), jnp.int32),
          pltpu.VMEM((K,), jnp.float32),
          pltpu.VMEM((STRIPE,), jnp.float32),
          pltpu.VMEM((STRIPE,), jnp.float32),
          pltpu.VMEM((N_PAD,), jnp.float32),
          pltpu.VMEM((K, D), jnp.float32),
          pltpu.VMEM((K + 16,), jnp.float32),
          pltpu.VMEM((16, D), jnp.float32),
          pltpu.VMEM_SHARED((N_PAD, D), jnp.float32),
          pltpu.SemaphoreType.DMA,
      ],
  )
  def _sc_pass_c(x_hbm, sidx_hbm, ridx_hbm, ex_hbm, denom_hbm, agg_hbm,
                 sidxc_v, ridxc_v, exc_v, d0_v, d1_v, inv_v,
                 rows_v, attn_v, zb_v, agg_sp, sem):
    cid = lax.axis_index("c")
    sid = lax.axis_index("s")
    wid = sid * NC + cid

    def _inv(i, carry):
        base = i * STRIPE
        pltpu.sync_copy(denom_hbm.at[0].at[pl.ds(base, STRIPE)], d0_v)
        pltpu.sync_copy(denom_hbm.at[1].at[pl.ds(base, STRIPE)], d1_v)
        def _inv16(j, carry2):
            sl = pl.ds(j * 16, 16)
            inv_v[pl.ds(base + j * 16, 16)] = 1.0 / jnp.maximum(
                d0_v[sl] + d1_v[sl], 1e-30)
            return carry2
        lax.fori_loop(0, STRIPE // 16, _inv16, 0)
        return carry
    lax.fori_loop(0, N_PAD // STRIPE, _inv, 0)

    for u in range(16):
        for v in range(D // 16):
            zb_v[u, pl.ds(v * 16, 16)] = jnp.zeros((16,), jnp.float32)

    def _zero(i, carry):
        pltpu.sync_copy(zb_v, agg_sp.at[pl.ds(sid * STRIPE + i * 16, 16)])
        return carry
    lax.fori_loop(0, STRIPE // 16, _zero, 0)
    plsc.subcore_barrier()

    def _chunk(cc, carry):
        base = cc * K
        pltpu.sync_copy(sidx_hbm.at[wid].at[pl.ds(base, K)], sidxc_v)
        pltpu.sync_copy(ridx_hbm.at[wid].at[pl.ds(base, K)], ridxc_v)
        pltpu.sync_copy(ex_hbm.at[wid].at[pl.ds(base, K)], exc_v)
        pltpu.async_copy(x_hbm.at[sidxc_v], rows_v, sem).wait()
        for v in range(K // 16):
            sl = pl.ds(v * 16, 16)
            ri = ridxc_v[sl]
            invv = plsc.load_gather(inv_v, [ri])
            attn_v[sl] = exc_v[sl] * invv

        def _rowscale(j, carry2):
            a = attn_v[pl.ds(j, 16)][0]
            for u in range(D // 16):
                sl = pl.ds(u * 16, 16)
                rows_v[j, sl] = rows_v[j, sl] * a
            return carry2
        lax.fori_loop(0, K, _rowscale, 0)
        pltpu.sync_copy(rows_v, agg_sp.at[ridxc_v], add=True)
        return carry
    lax.fori_loop(0, NCHUNK, _chunk, 0)
    plsc.subcore_barrier()

    def _out(i, carry):
        row = sid * STRIPE + i * 16
        pltpu.sync_copy(agg_sp.at[pl.ds(row, 16)], zb_v)
        pltpu.sync_copy(zb_v, agg_hbm.at[cid].at[pl.ds(row, 16)])
        return carry
    lax.fori_loop(0, STRIPE // 16, _out, 0)

  return _sc_pass_c


def kernel(nodes, senders, receivers, enc_w, enc_b, attn0_w, attn0_b, core0_w,
           core0_b, attn1_w, attn1_b, core1_w, core1_b, dec_w, dec_b):
    nodes_p = jnp.pad(nodes, ((0, N_PAD - N), (0, 0)))
    s_idx, r_idx = receivers, senders
    s2 = jnp.pad(s_idx.reshape(NW, E // NW), ((0, 0), (0, EPT - E // NW)),
                 constant_values=0)
    r2 = jnp.pad(r_idx.reshape(NW, E // NW), ((0, 0), (0, EPT - E // NW)),
                 constant_values=N)
    r3 = r2.reshape(NW, NCHUNK, K)

    x, scores = _tc_encode(nodes_p, enc_w, enc_b, attn0_w, attn0_b)
    for hop, (aw, ab, cw, cb) in enumerate(
            ((attn0_w, attn0_b, core0_w, core0_b),
             (attn1_w, attn1_b, core1_w, core1_b))):
        ex, denom = _make_sc_pass_a()(scores.reshape(2 * N_PAD), s2, r2, r3)
        (agg2,) = _make_sc_pass_c()(x, s2, r2, ex, denom)
        if hop == 0:
            x, scores = _tc_update(agg2, cw, cb, attn1_w, attn1_b)
        else:
            out = _tc_decode(agg2, cw, cb, dec_w, dec_b)
    return out[:N]


# speedup vs baseline: 13.4400x; 1.1743x over previous
import functools

import jax
import jax.numpy as jnp
from jax import lax
from jax.experimental import pallas as pl
from jax.experimental.pallas import tpu as pltpu
from jax.experimental.pallas import tpu_sc as plsc

N = 10000
E = 320000
D = 128
C = 40
NEG_SLOPE = 0.2

NC = 2
NS = 16
NW = NC * NS
N_PAD = 10240
EPT = E // NW + 240
assert EPT == 10240
K = 128
NCHUNK = EPT // K
STRIPE = N_PAD // NS


def _tc1_body(nodes_ref, ew_ref, eb_ref, aw_ref, ab_ref, x_ref, sc_ref):
    x = jnp.maximum(jnp.dot(nodes_ref[...], ew_ref[...],
                            preferred_element_type=jnp.float32)
                    + eb_ref[...][None, :], 0.0)
    x_ref[...] = x
    a_s = jnp.dot(x, aw_ref[...][:D, :], preferred_element_type=jnp.float32)
    a_d = jnp.dot(x, aw_ref[...][D:, :], preferred_element_type=jnp.float32)
    a_d = a_d + ab_ref[...][None, :]
    sc_ref[...] = jnp.concatenate([a_s, a_d], axis=1)


def _tc2_body(agg_ref, cw_ref, cb_ref, aw_ref, ab_ref, x_ref, sc_ref):
    agg = agg_ref[0] + agg_ref[1]
    x = jnp.maximum(jnp.dot(agg, cw_ref[...],
                            preferred_element_type=jnp.float32)
                    + cb_ref[...][None, :], 0.0) + agg
    x_ref[...] = x
    a_s = jnp.dot(x, aw_ref[...][:D, :], preferred_element_type=jnp.float32)
    a_d = jnp.dot(x, aw_ref[...][D:, :], preferred_element_type=jnp.float32)
    a_d = a_d + ab_ref[...][None, :]
    sc_ref[...] = jnp.concatenate([a_s, a_d], axis=1)


def _tc3_body(agg_ref, cw_ref, cb_ref, dw_ref, db_ref, out_ref):
    agg = agg_ref[0] + agg_ref[1]
    x = jnp.maximum(jnp.dot(agg, cw_ref[...],
                            preferred_element_type=jnp.float32)
                    + cb_ref[...][None, :], 0.0) + agg
    out_ref[...] = jnp.dot(x, dw_ref[...],
                           preferred_element_type=jnp.float32) + db_ref[...][None, :]


_BLK = 1024


def _tc_encode(nodes_p, enc_w, enc_b, aw, ab):
    return pl.pallas_call(
        _tc1_body,
        grid=(N_PAD // _BLK,),
        in_specs=[
            pl.BlockSpec((_BLK, D), lambda i: (i, 0)),
            pl.BlockSpec((D, D), lambda i: (0, 0)),
            pl.BlockSpec((D,), lambda i: (0,)),
            pl.BlockSpec((2 * D, 1), lambda i: (0, 0)),
            pl.BlockSpec((1,), lambda i: (0,)),
        ],
        out_specs=[
            pl.BlockSpec((_BLK, D), lambda i: (i, 0)),
            pl.BlockSpec((_BLK, 2), lambda i: (i, 0)),
        ],
        out_shape=[
            jax.ShapeDtypeStruct((N_PAD, D), jnp.float32),
            jax.ShapeDtypeStruct((N_PAD, 2), jnp.float32),
        ],
    )(nodes_p, enc_w, enc_b, aw, ab)


def _tc_update(agg2, cw, cb, aw, ab):
    return pl.pallas_call(
        _tc2_body,
        grid=(N_PAD // _BLK,),
        in_specs=[
            pl.BlockSpec((2, _BLK, D), lambda i: (0, i, 0)),
            pl.BlockSpec((D, D), lambda i: (0, 0)),
            pl.BlockSpec((D,), lambda i: (0,)),
            pl.BlockSpec((2 * D, 1), lambda i: (0, 0)),
            pl.BlockSpec((1,), lambda i: (0,)),
        ],
        out_specs=[
            pl.BlockSpec((_BLK, D), lambda i: (i, 0)),
            pl.BlockSpec((_BLK, 2), lambda i: (i, 0)),
        ],
        out_shape=[
            jax.ShapeDtypeStruct((N_PAD, D), jnp.float32),
            jax.ShapeDtypeStruct((N_PAD, 2), jnp.float32),
        ],
    )(agg2, cw, cb, aw, ab)


def _tc_decode(agg2, cw, cb, dw, db):
    return pl.pallas_call(
        _tc3_body,
        grid=(N_PAD // _BLK,),
        in_specs=[
            pl.BlockSpec((2, _BLK, D), lambda i: (0, i, 0)),
            pl.BlockSpec((D, D), lambda i: (0, 0)),
            pl.BlockSpec((D,), lambda i: (0,)),
            pl.BlockSpec((D, C), lambda i: (0, 0)),
            pl.BlockSpec((C,), lambda i: (0,)),
        ],
        out_specs=pl.BlockSpec((_BLK, C), lambda i: (i, 0)),
        out_shape=jax.ShapeDtypeStruct((N_PAD, C), jnp.float32),
    )(agg2, cw, cb, dw, db)


@functools.cache
def _make_sc_pass_a():
  mesh = plsc.VectorSubcoreMesh(core_axis_name="c", subcore_axis_name="s")

  @functools.partial(
      pl.kernel,
      mesh=mesh,
      out_type=[
          jax.ShapeDtypeStruct((NW, EPT), jnp.float32),
          jax.ShapeDtypeStruct((NC, N_PAD), jnp.float32),
      ],
      compiler_params=pltpu.CompilerParams(needs_layout_passes=False),
      scratch_types=[
          pltpu.VMEM((2 * N_PAD,), jnp.float32),
          pltpu.VMEM((EPT,), jnp.int32),
          pltpu.VMEM((EPT,), jnp.int32),
          pltpu.VMEM((NCHUNK, K), jnp.int32),
          pltpu.VMEM((EPT,), jnp.float32),
          pltpu.VMEM((STRIPE,), jnp.float32),
          pltpu.VMEM_SHARED((N_PAD,), jnp.float32),
      ],
  )
  def _sc_pass_a(scores_hbm, sidx_hbm, ridx_hbm, ridx2_hbm, ex_hbm, denom_hbm,
                 scores_v, sidx_v, ridxv_v, ridx2_v, ex_v, zb_v, denom_sp):
    cid = lax.axis_index("c")
    sid = lax.axis_index("s")
    wid = sid * NC + cid

    pltpu.sync_copy(scores_hbm, scores_v)
    pltpu.sync_copy(sidx_hbm.at[wid], sidx_v)
    pltpu.sync_copy(ridx_hbm.at[wid], ridxv_v)
    pltpu.sync_copy(ridx2_hbm.at[wid], ridx2_v)

    def _zero(i, carry):
        zb_v[pl.ds(i * 16, 16)] = jnp.zeros((16,), jnp.float32)
        return carry
    lax.fori_loop(0, STRIPE // 16, _zero, 0)
    pltpu.sync_copy(zb_v, denom_sp.at[pl.ds(sid * STRIPE, STRIPE)])
    plsc.subcore_barrier()

    def _chunk(cc, carry):
        base = cc * K
        for v in range(K // 16):
            off = base + v * 16
            si = sidx_v[pl.ds(off, 16)]
            ri = ridxv_v[pl.ds(off, 16)]
            a_s = plsc.load_gather(scores_v, [si * 2])
            a_d = plsc.load_gather(scores_v, [ri * 2 + 1])
            t = a_s + a_d
            nl = jnp.maximum(t, t * NEG_SLOPE)
            ex_v[pl.ds(off, 16)] = jnp.exp(nl)
        pltpu.sync_copy(ex_v.at[pl.ds(base, K)],
                        denom_sp.at[ridx2_v.at[cc]], add=True)
        return carry
    lax.fori_loop(0, NCHUNK, _chunk, 0)

    pltpu.sync_copy(ex_v, ex_hbm.at[wid])
    plsc.subcore_barrier()
    pltpu.sync_copy(denom_sp.at[pl.ds(sid * STRIPE, STRIPE)], zb_v)
    pltpu.sync_copy(zb_v, denom_hbm.at[cid].at[pl.ds(sid * STRIPE, STRIPE)])

  return _sc_pass_a


@functools.cache
def _make_sc_pass_c():
  mesh = plsc.VectorSubcoreMesh(core_axis_name="c", subcore_axis_name="s")

  @functools.partial(
      pl.kernel,
      mesh=mesh,
      out_type=[
          jax.ShapeDtypeStruct((NC, N_PAD, D), jnp.float32),
      ],
      compiler_params=pltpu.CompilerParams(needs_layout_passes=False),
      scratch_types=[
          pltpu.VMEM((2, K), jnp.int32),
          pltpu.VMEM((2, K), jnp.int32),
          pltpu.VMEM((2, K), jnp.float32),
          pltpu.VMEM((STRIPE,), jnp.float32),
          pltpu.VMEM((STRIPE,), jnp.float32),
          pltpu.VMEM((N_PAD,), jnp.float32),
          pltpu.VMEM((K, D), jnp.float32),
          pltpu.VMEM((K + 16,), jnp.float32),
          pltpu.VMEM((16, D), jnp.float32),
          pltpu.VMEM_SHARED((N_PAD, D), jnp.float32),
          pltpu.SemaphoreType.DMA,
          pltpu.SemaphoreType.DMA,
      ],
  )
  def _sc_pass_c(x_hbm, sidx_hbm, ridx_hbm, ex_hbm, denom_hbm, agg_hbm,
                 sidxc_v, ridxc_v, exc_v, d0_v, d1_v, inv_v,
                 rows_v, attn_v, zb_v, agg_sp, semlin, semg):
    cid = lax.axis_index("c")
    sid = lax.axis_index("s")
    wid = sid * NC + cid

    def _fire_lin(c, b):
        sl = pl.ds(c * K, K)
        pltpu.async_copy(sidx_hbm.at[wid].at[sl], sidxc_v.at[b], semlin)
        pltpu.async_copy(ridx_hbm.at[wid].at[sl], ridxc_v.at[b], semlin)
        pltpu.async_copy(ex_hbm.at[wid].at[sl], exc_v.at[b], semlin)

    def _drain_lin(b):
        sl = pl.ds(0, K)
        pltpu.make_async_copy(sidx_hbm.at[wid].at[sl], sidxc_v.at[b],
                              semlin).wait()
        pltpu.make_async_copy(ridx_hbm.at[wid].at[sl], ridxc_v.at[b],
                              semlin).wait()
        pltpu.make_async_copy(ex_hbm.at[wid].at[sl], exc_v.at[b],
                              semlin).wait()

    def _inv(i, carry):
        base = i * STRIPE
        pltpu.sync_copy(denom_hbm.at[0].at[pl.ds(base, STRIPE)], d0_v)
        pltpu.sync_copy(denom_hbm.at[1].at[pl.ds(base, STRIPE)], d1_v)

        def _inv16(j, carry2):
            sl = pl.ds(j * 16, 16)
            inv_v[pl.ds(base + j * 16, 16)] = 1.0 / jnp.maximum(
                d0_v[sl] + d1_v[sl], 1e-30)
            return carry2
        lax.fori_loop(0, STRIPE // 16, _inv16, 0)
        return carry
    lax.fori_loop(0, N_PAD // STRIPE, _inv, 0)

    for u in range(16):
        for v in range(D // 16):
            zb_v[u, pl.ds(v * 16, 16)] = jnp.zeros((16,), jnp.float32)

    def _zero(i, carry):
        pltpu.sync_copy(zb_v, agg_sp.at[pl.ds(sid * STRIPE + i * 16, 16)])
        return carry
    lax.fori_loop(0, STRIPE // 16, _zero, 0)
    plsc.subcore_barrier()

    _fire_lin(0, 0)

    def _step(c, b):
        _drain_lin(b)
        _fire_lin(jnp.minimum(c + 1, NCHUNK - 1), 1 - b)
        g = pltpu.async_copy(x_hbm.at[sidxc_v.at[b]], rows_v, semg)
        for v in range(K // 16):
            sl = pl.ds(v * 16, 16)
            ri = ridxc_v[b, sl]
            invv = plsc.load_gather(inv_v, [ri])
            attn_v[sl] = exc_v[b, sl] * invv
        g.wait()

        def _rowscale(j, carry2):
            a = attn_v[pl.ds(j, 16)][0]
            for u in range(D // 16):
                sl = pl.ds(u * 16, 16)
                rows_v[j, sl] = rows_v[j, sl] * a
            return carry2
        lax.fori_loop(0, K, _rowscale, 0, unroll=4)
        pltpu.sync_copy(rows_v, agg_sp.at[ridxc_v.at[b]], add=True)

    def _pair(oo, carry):
        _step(oo * 2, 0)
        _step(oo * 2 + 1, 1)
        return carry
    lax.fori_loop(0, NCHUNK // 2, _pair, 0)
    _drain_lin(0)
    plsc.subcore_barrier()

    def _out(i, carry):
        row = sid * STRIPE + i * 16
        pltpu.sync_copy(agg_sp.at[pl.ds(row, 16)], zb_v)
        pltpu.sync_copy(zb_v, agg_hbm.at[cid].at[pl.ds(row, 16)])
        return carry
    lax.fori_loop(0, STRIPE // 16, _out, 0)

  return _sc_pass_c


def kernel(nodes, senders, receivers, enc_w, enc_b, attn0_w, attn0_b, core0_w,
           core0_b, attn1_w, attn1_b, core1_w, core1_b, dec_w, dec_b):
    nodes_p = jnp.pad(nodes, ((0, N_PAD - N), (0, 0)))
    s_idx, r_idx = receivers, senders
    s2 = jnp.pad(s_idx.reshape(NW, E // NW), ((0, 0), (0, EPT - E // NW)),
                 constant_values=0)
    r2 = jnp.pad(r_idx.reshape(NW, E // NW), ((0, 0), (0, EPT - E // NW)),
                 constant_values=N)
    r3 = r2.reshape(NW, NCHUNK, K)

    x, scores = _tc_encode(nodes_p, enc_w, enc_b, attn0_w, attn0_b)
    for hop, (aw, ab, cw, cb) in enumerate(
            ((attn0_w, attn0_b, core0_w, core0_b),
             (attn1_w, attn1_b, core1_w, core1_b))):
        ex, denom = _make_sc_pass_a()(scores.reshape(2 * N_PAD), s2, r2, r3)
        (agg2,) = _make_sc_pass_c()(x, s2, r2, ex, denom)
        if hop == 0:
            x, scores = _tc_update(agg2, cw, cb, attn1_w, attn1_b)
        else:
            out = _tc_decode(agg2, cw, cb, dec_w, dec_b)
    return out[:N]


# speedup vs baseline: 13.4832x; 1.0032x over previous
import functools

import jax
import jax.numpy as jnp
from jax import lax
from jax.experimental import pallas as pl
from jax.experimental.pallas import tpu as pltpu
from jax.experimental.pallas import tpu_sc as plsc

N = 10000
E = 320000
D = 128
C = 40
NEG_SLOPE = 0.2

NC = 2
NS = 16
NW = NC * NS
N_PAD = 10240
EPT = E // NW + 240
assert EPT == 10240
K = 128
NCHUNK = EPT // K
STRIPE = N_PAD // NS


def _tc1_body(nodes_ref, ew_ref, eb_ref, aw_ref, ab_ref, x_ref, sc_ref):
    x = jnp.maximum(jnp.dot(nodes_ref[...], ew_ref[...],
                            preferred_element_type=jnp.float32)
                    + eb_ref[...][None, :], 0.0)
    x_ref[...] = x
    a_s = jnp.dot(x, aw_ref[...][:D, :], preferred_element_type=jnp.float32)
    a_d = jnp.dot(x, aw_ref[...][D:, :], preferred_element_type=jnp.float32)
    a_d = a_d + ab_ref[...][None, :]
    sc_ref[...] = jnp.concatenate([a_s, a_d], axis=1)


def _tc2_body(agg_ref, cw_ref, cb_ref, aw_ref, ab_ref, x_ref, sc_ref):
    agg = agg_ref[0] + agg_ref[1]
    x = jnp.maximum(jnp.dot(agg, cw_ref[...],
                            preferred_element_type=jnp.float32)
                    + cb_ref[...][None, :], 0.0) + agg
    x_ref[...] = x
    a_s = jnp.dot(x, aw_ref[...][:D, :], preferred_element_type=jnp.float32)
    a_d = jnp.dot(x, aw_ref[...][D:, :], preferred_element_type=jnp.float32)
    a_d = a_d + ab_ref[...][None, :]
    sc_ref[...] = jnp.concatenate([a_s, a_d], axis=1)


def _tc3_body(agg_ref, cw_ref, cb_ref, dw_ref, db_ref, out_ref):
    agg = agg_ref[0] + agg_ref[1]
    x = jnp.maximum(jnp.dot(agg, cw_ref[...],
                            preferred_element_type=jnp.float32)
                    + cb_ref[...][None, :], 0.0) + agg
    out_ref[...] = jnp.dot(x, dw_ref[...],
                           preferred_element_type=jnp.float32) + db_ref[...][None, :]


_BLK = 1024


def _tc_encode(nodes_p, enc_w, enc_b, aw, ab):
    return pl.pallas_call(
        _tc1_body,
        grid=(N_PAD // _BLK,),
        in_specs=[
            pl.BlockSpec((_BLK, D), lambda i: (i, 0)),
            pl.BlockSpec((D, D), lambda i: (0, 0)),
            pl.BlockSpec((D,), lambda i: (0,)),
            pl.BlockSpec((2 * D, 1), lambda i: (0, 0)),
            pl.BlockSpec((1,), lambda i: (0,)),
        ],
        out_specs=[
            pl.BlockSpec((_BLK, D), lambda i: (i, 0)),
            pl.BlockSpec((_BLK, 2), lambda i: (i, 0)),
        ],
        out_shape=[
            jax.ShapeDtypeStruct((N_PAD, D), jnp.float32),
            jax.ShapeDtypeStruct((N_PAD, 2), jnp.float32),
        ],
    )(nodes_p, enc_w, enc_b, aw, ab)


def _tc_update(agg2, cw, cb, aw, ab):
    return pl.pallas_call(
        _tc2_body,
        grid=(N_PAD // _BLK,),
        in_specs=[
            pl.BlockSpec((2, _BLK, D), lambda i: (0, i, 0)),
            pl.BlockSpec((D, D), lambda i: (0, 0)),
            pl.BlockSpec((D,), lambda i: (0,)),
            pl.BlockSpec((2 * D, 1), lambda i: (0, 0)),
            pl.BlockSpec((1,), lambda i: (0,)),
        ],
        out_specs=[
            pl.BlockSpec((_BLK, D), lambda i: (i, 0)),
            pl.BlockSpec((_BLK, 2), lambda i: (i, 0)),
        ],
        out_shape=[
            jax.ShapeDtypeStruct((N_PAD, D), jnp.float32),
            jax.ShapeDtypeStruct((N_PAD, 2), jnp.float32),
        ],
    )(agg2, cw, cb, aw, ab)


def _tc_decode(agg2, cw, cb, dw, db):
    return pl.pallas_call(
        _tc3_body,
        grid=(N_PAD // _BLK,),
        in_specs=[
            pl.BlockSpec((2, _BLK, D), lambda i: (0, i, 0)),
            pl.BlockSpec((D, D), lambda i: (0, 0)),
            pl.BlockSpec((D,), lambda i: (0,)),
            pl.BlockSpec((D, C), lambda i: (0, 0)),
            pl.BlockSpec((C,), lambda i: (0,)),
        ],
        out_specs=pl.BlockSpec((_BLK, C), lambda i: (i, 0)),
        out_shape=jax.ShapeDtypeStruct((N_PAD, C), jnp.float32),
    )(agg2, cw, cb, dw, db)


@functools.cache
def _make_sc_pass_a():
  mesh = plsc.VectorSubcoreMesh(core_axis_name="c", subcore_axis_name="s")

  @functools.partial(
      pl.kernel,
      mesh=mesh,
      out_type=[
          jax.ShapeDtypeStruct((NW, EPT), jnp.float32),
          jax.ShapeDtypeStruct((NC, N_PAD), jnp.float32),
      ],
      compiler_params=pltpu.CompilerParams(needs_layout_passes=False),
      scratch_types=[
          pltpu.VMEM((2 * N_PAD,), jnp.float32),
          pltpu.VMEM((EPT,), jnp.int32),
          pltpu.VMEM((EPT,), jnp.int32),
          pltpu.VMEM((NCHUNK, K), jnp.int32),
          pltpu.VMEM((EPT,), jnp.float32),
          pltpu.VMEM((STRIPE,), jnp.float32),
          pltpu.VMEM_SHARED((N_PAD,), jnp.float32),
      ],
  )
  def _sc_pass_a(scores_hbm, sidx_hbm, ridx_hbm, ridx2_hbm, ex_hbm, denom_hbm,
                 scores_v, sidx_v, ridxv_v, ridx2_v, ex_v, zb_v, denom_sp):
    cid = lax.axis_index("c")
    sid = lax.axis_index("s")
    wid = sid * NC + cid

    pltpu.sync_copy(scores_hbm, scores_v)
    pltpu.sync_copy(sidx_hbm.at[wid], sidx_v)
    pltpu.sync_copy(ridx_hbm.at[wid], ridxv_v)
    pltpu.sync_copy(ridx2_hbm.at[wid], ridx2_v)

    def _zero(i, carry):
        zb_v[pl.ds(i * 16, 16)] = jnp.zeros((16,), jnp.float32)
        return carry
    lax.fori_loop(0, STRIPE // 16, _zero, 0)
    pltpu.sync_copy(zb_v, denom_sp.at[pl.ds(sid * STRIPE, STRIPE)])
    plsc.subcore_barrier()

    def _chunk(cc, carry):
        base = cc * K
        for v in range(K // 16):
            off = base + v * 16
            si = sidx_v[pl.ds(off, 16)]
            ri = ridxv_v[pl.ds(off, 16)]
            a_s = plsc.load_gather(scores_v, [si * 2])
            a_d = plsc.load_gather(scores_v, [ri * 2 + 1])
            t = a_s + a_d
            nl = jnp.maximum(t, t * NEG_SLOPE)
            ex_v[pl.ds(off, 16)] = jnp.exp(nl)
        pltpu.sync_copy(ex_v.at[pl.ds(base, K)],
                        denom_sp.at[ridx2_v.at[cc]], add=True)
        return carry
    lax.fori_loop(0, NCHUNK, _chunk, 0)

    pltpu.sync_copy(ex_v, ex_hbm.at[wid])
    plsc.subcore_barrier()
    pltpu.sync_copy(denom_sp.at[pl.ds(sid * STRIPE, STRIPE)], zb_v)
    pltpu.sync_copy(zb_v, denom_hbm.at[cid].at[pl.ds(sid * STRIPE, STRIPE)])

  return _sc_pass_a


@functools.cache
def _make_sc_pass_c():
  mesh = plsc.VectorSubcoreMesh(core_axis_name="c", subcore_axis_name="s")
  HK = K // 2

  @functools.partial(
      pl.kernel,
      mesh=mesh,
      out_type=[
          jax.ShapeDtypeStruct((NC, N_PAD, D), jnp.float32),
      ],
      compiler_params=pltpu.CompilerParams(needs_layout_passes=False),
      scratch_types=[
          pltpu.VMEM((2, K), jnp.int32),
          pltpu.VMEM((2, K), jnp.int32),
          pltpu.VMEM((2, K), jnp.float32),
          pltpu.VMEM((2, HK), jnp.int32),
          pltpu.VMEM((STRIPE,), jnp.float32),
          pltpu.VMEM((STRIPE,), jnp.float32),
          pltpu.VMEM((N_PAD,), jnp.float32),
          pltpu.VMEM((2, HK, D), jnp.float32),
          pltpu.VMEM((K,), jnp.float32),
          pltpu.VMEM((16, D), jnp.float32),
          pltpu.VMEM_SHARED((N_PAD, D), jnp.float32),
          pltpu.SemaphoreType.DMA,
          pltpu.SemaphoreType.DMA,
          pltpu.SemaphoreType.DMA,
      ],
  )
  def _sc_pass_c(x_hbm, sidx_hbm, ridx_hbm, ex_hbm, denom_hbm, agg_hbm,
                 sidxc_v, ridxc_v, exc_v, ridxs_v, d0_v, d1_v, inv_v,
                 rows_v, attn_v, zb_v, agg_sp, semlin, semga, semgb):
    cid = lax.axis_index("c")
    sid = lax.axis_index("s")
    wid = sid * NC + cid

    def _fire_lin(c, b):
        sl = pl.ds(c * K, K)
        pltpu.async_copy(sidx_hbm.at[wid].at[sl], sidxc_v.at[b], semlin)
        pltpu.async_copy(ridx_hbm.at[wid].at[sl], ridxc_v.at[b], semlin)
        pltpu.async_copy(ex_hbm.at[wid].at[sl], exc_v.at[b], semlin)

    def _drain_lin(b):
        sl = pl.ds(0, K)
        pltpu.make_async_copy(sidx_hbm.at[wid].at[sl], sidxc_v.at[b],
                              semlin).wait()
        pltpu.make_async_copy(ridx_hbm.at[wid].at[sl], ridxc_v.at[b],
                              semlin).wait()
        pltpu.make_async_copy(ex_hbm.at[wid].at[sl], exc_v.at[b],
                              semlin).wait()

    def _inv(i, carry):
        base = i * STRIPE
        pltpu.sync_copy(denom_hbm.at[0].at[pl.ds(base, STRIPE)], d0_v)
        pltpu.sync_copy(denom_hbm.at[1].at[pl.ds(base, STRIPE)], d1_v)

        def _inv16(j, carry2):
            sl = pl.ds(j * 16, 16)
            inv_v[pl.ds(base + j * 16, 16)] = 1.0 / jnp.maximum(
                d0_v[sl] + d1_v[sl], 1e-30)
            return carry2
        lax.fori_loop(0, STRIPE // 16, _inv16, 0)
        return carry
    lax.fori_loop(0, N_PAD // STRIPE, _inv, 0)

    for u in range(16):
        for v in range(D // 16):
            zb_v[u, pl.ds(v * 16, 16)] = jnp.zeros((16,), jnp.float32)

    def _zero(i, carry):
        pltpu.sync_copy(zb_v, agg_sp.at[pl.ds(sid * STRIPE + i * 16, 16)])
        return carry
    lax.fori_loop(0, STRIPE // 16, _zero, 0)
    plsc.subcore_barrier()

    _fire_lin(0, 0)

    def _half(b, h, g):
        g.wait()

        def _rowscale(j, carry2):
            av = plsc.load_gather(attn_v, [jnp.zeros((16,), jnp.int32)
                                           + (j + h * HK)])
            for u in range(D // 16):
                sl = pl.ds(u * 16, 16)
                rows_v[h, j, sl] = rows_v[h, j, sl] * av
            return carry2
        lax.fori_loop(0, HK, _rowscale, 0, unroll=4)
        pltpu.sync_copy(rows_v.at[h], agg_sp.at[ridxs_v.at[h]], add=True)

    def _step(c, b):
        _drain_lin(b)
        _fire_lin(jnp.minimum(c + 1, NCHUNK - 1), 1 - b)
        ga = pltpu.async_copy(x_hbm.at[sidxc_v.at[b].at[pl.ds(0, HK)]],
                              rows_v.at[0], semga)
        gb = pltpu.async_copy(x_hbm.at[sidxc_v.at[b].at[pl.ds(HK, HK)]],
                              rows_v.at[1], semgb)
        for h in range(2):
            for v in range(HK // 16):
                ridxs_v[h, pl.ds(v * 16, 16)] = (
                    ridxc_v[b, pl.ds(h * HK + v * 16, 16)])
        for v in range(K // 16):
            sl = pl.ds(v * 16, 16)
            ri = ridxc_v[b, sl]
            invv = plsc.load_gather(inv_v, [ri])
            attn_v[sl] = exc_v[b, sl] * invv
        _half(b, 0, ga)
        _half(b, 1, gb)

    def _pair(oo, carry):
        _step(oo * 2, 0)
        _step(oo * 2 + 1, 1)
        return carry
    lax.fori_loop(0, NCHUNK // 2, _pair, 0)
    _drain_lin(0)
    plsc.subcore_barrier()

    def _out(i, carry):
        row = sid * STRIPE + i * 16
        pltpu.sync_copy(agg_sp.at[pl.ds(row, 16)], zb_v)
        pltpu.sync_copy(zb_v, agg_hbm.at[cid].at[pl.ds(row, 16)])
        return carry
    lax.fori_loop(0, STRIPE // 16, _out, 0)

  return _sc_pass_c


def kernel(nodes, senders, receivers, enc_w, enc_b, attn0_w, attn0_b, core0_w,
           core0_b, attn1_w, attn1_b, core1_w, core1_b, dec_w, dec_b):
    nodes_p = jnp.pad(nodes, ((0, N_PAD - N), (0, 0)))
    s_idx, r_idx = receivers, senders
    s2 = jnp.pad(s_idx.reshape(NW, E // NW), ((0, 0), (0, EPT - E // NW)),
                 constant_values=0)
    r2 = jnp.pad(r_idx.reshape(NW, E // NW), ((0, 0), (0, EPT - E // NW)),
                 constant_values=N)
    r3 = r2.reshape(NW, NCHUNK, K)

    x, scores = _tc_encode(nodes_p, enc_w, enc_b, attn0_w, attn0_b)
    for hop, (aw, ab, cw, cb) in enumerate(
            ((attn0_w, attn0_b, core0_w, core0_b),
             (attn1_w, attn1_b, core1_w, core1_b))):
        ex, denom = _make_sc_pass_a()(scores.reshape(2 * N_PAD), s2, r2, r3)
        (agg2,) = _make_sc_pass_c()(x, s2, r2, ex, denom)
        if hop == 0:
            x, scores = _tc_update(agg2, cw, cb, attn1_w, attn1_b)
        else:
            out = _tc_decode(agg2, cw, cb, dec_w, dec_b)
    return out[:N]


# speedup vs baseline: 14.3261x; 1.0625x over previous
import functools

import jax
import jax.numpy as jnp
from jax import lax
from jax.experimental import pallas as pl
from jax.experimental.pallas import tpu as pltpu
from jax.experimental.pallas import tpu_sc as plsc

N = 10000
E = 320000
D = 128
C = 40
NEG_SLOPE = 0.2

NC = 2
NS = 16
NW = NC * NS
N_PAD = 10240
EPT = E // NW + 240
assert EPT == 10240
K = 128
NCHUNK = EPT // K
STRIPE = N_PAD // NS


def _tc1_body(nodes_ref, ew_ref, eb_ref, aw_ref, ab_ref, x_ref, sc_ref):
    x = jnp.maximum(jnp.dot(nodes_ref[...], ew_ref[...],
                            preferred_element_type=jnp.float32)
                    + eb_ref[...][None, :], 0.0)
    x_ref[...] = x
    a_s = jnp.dot(x, aw_ref[...][:D, :], preferred_element_type=jnp.float32)
    a_d = jnp.dot(x, aw_ref[...][D:, :], preferred_element_type=jnp.float32)
    a_d = a_d + ab_ref[...][None, :]
    sc_ref[...] = jnp.concatenate([a_s, a_d], axis=1)


def _tc2_body(agg_ref, cw_ref, cb_ref, aw_ref, ab_ref, x_ref, sc_ref):
    agg = agg_ref[0] + agg_ref[1]
    x = jnp.maximum(jnp.dot(agg, cw_ref[...],
                            preferred_element_type=jnp.float32)
                    + cb_ref[...][None, :], 0.0) + agg
    x_ref[...] = x
    a_s = jnp.dot(x, aw_ref[...][:D, :], preferred_element_type=jnp.float32)
    a_d = jnp.dot(x, aw_ref[...][D:, :], preferred_element_type=jnp.float32)
    a_d = a_d + ab_ref[...][None, :]
    sc_ref[...] = jnp.concatenate([a_s, a_d], axis=1)


def _tc3_body(agg_ref, cw_ref, cb_ref, dw_ref, db_ref, out_ref):
    agg = agg_ref[0] + agg_ref[1]
    x = jnp.maximum(jnp.dot(agg, cw_ref[...],
                            preferred_element_type=jnp.float32)
                    + cb_ref[...][None, :], 0.0) + agg
    out_ref[...] = jnp.dot(x, dw_ref[...],
                           preferred_element_type=jnp.float32) + db_ref[...][None, :]


_BLK = 1024


def _tc_encode(nodes_p, enc_w, enc_b, aw, ab):
    return pl.pallas_call(
        _tc1_body,
        grid=(N_PAD // _BLK,),
        in_specs=[
            pl.BlockSpec((_BLK, D), lambda i: (i, 0)),
            pl.BlockSpec((D, D), lambda i: (0, 0)),
            pl.BlockSpec((D,), lambda i: (0,)),
            pl.BlockSpec((2 * D, 1), lambda i: (0, 0)),
            pl.BlockSpec((1,), lambda i: (0,)),
        ],
        out_specs=[
            pl.BlockSpec((_BLK, D), lambda i: (i, 0)),
            pl.BlockSpec((_BLK, 2), lambda i: (i, 0)),
        ],
        out_shape=[
            jax.ShapeDtypeStruct((N_PAD, D), jnp.float32),
            jax.ShapeDtypeStruct((N_PAD, 2), jnp.float32),
        ],
    )(nodes_p, enc_w, enc_b, aw, ab)


def _tc_update(agg2, cw, cb, aw, ab):
    return pl.pallas_call(
        _tc2_body,
        grid=(N_PAD // _BLK,),
        in_specs=[
            pl.BlockSpec((2, _BLK, D), lambda i: (0, i, 0)),
            pl.BlockSpec((D, D), lambda i: (0, 0)),
            pl.BlockSpec((D,), lambda i: (0,)),
            pl.BlockSpec((2 * D, 1), lambda i: (0, 0)),
            pl.BlockSpec((1,), lambda i: (0,)),
        ],
        out_specs=[
            pl.BlockSpec((_BLK, D), lambda i: (i, 0)),
            pl.BlockSpec((_BLK, 2), lambda i: (i, 0)),
        ],
        out_shape=[
            jax.ShapeDtypeStruct((N_PAD, D), jnp.float32),
            jax.ShapeDtypeStruct((N_PAD, 2), jnp.float32),
        ],
    )(agg2, cw, cb, aw, ab)


def _tc_decode(agg2, cw, cb, dw, db):
    return pl.pallas_call(
        _tc3_body,
        grid=(N_PAD // _BLK,),
        in_specs=[
            pl.BlockSpec((2, _BLK, D), lambda i: (0, i, 0)),
            pl.BlockSpec((D, D), lambda i: (0, 0)),
            pl.BlockSpec((D,), lambda i: (0,)),
            pl.BlockSpec((D, C), lambda i: (0, 0)),
            pl.BlockSpec((C,), lambda i: (0,)),
        ],
        out_specs=pl.BlockSpec((_BLK, C), lambda i: (i, 0)),
        out_shape=jax.ShapeDtypeStruct((N_PAD, C), jnp.float32),
    )(agg2, cw, cb, dw, db)


@functools.cache
def _make_sc_pass_a():
  mesh = plsc.VectorSubcoreMesh(core_axis_name="c", subcore_axis_name="s")

  @functools.partial(
      pl.kernel,
      mesh=mesh,
      out_type=[
          jax.ShapeDtypeStruct((NW, EPT), jnp.float32),
          jax.ShapeDtypeStruct((NC, N_PAD), jnp.float32),
      ],
      compiler_params=pltpu.CompilerParams(needs_layout_passes=False),
      scratch_types=[
          pltpu.VMEM((2 * N_PAD,), jnp.float32),
          pltpu.VMEM((EPT,), jnp.int32),
          pltpu.VMEM((EPT,), jnp.int32),
          pltpu.VMEM((NCHUNK, K), jnp.int32),
          pltpu.VMEM((EPT,), jnp.float32),
          pltpu.VMEM((STRIPE,), jnp.float32),
          pltpu.VMEM_SHARED((N_PAD,), jnp.float32),
      ],
  )
  def _sc_pass_a(scores_hbm, sidx_hbm, ridx_hbm, ridx2_hbm, ex_hbm, denom_hbm,
                 scores_v, sidx_v, ridxv_v, ridx2_v, ex_v, zb_v, denom_sp):
    cid = lax.axis_index("c")
    sid = lax.axis_index("s")
    wid = sid * NC + cid

    pltpu.sync_copy(scores_hbm, scores_v)
    pltpu.sync_copy(sidx_hbm.at[wid], sidx_v)
    pltpu.sync_copy(ridx_hbm.at[wid], ridxv_v)
    pltpu.sync_copy(ridx2_hbm.at[wid], ridx2_v)

    def _zero(i, carry):
        zb_v[pl.ds(i * 16, 16)] = jnp.zeros((16,), jnp.float32)
        return carry
    lax.fori_loop(0, STRIPE // 16, _zero, 0)
    pltpu.sync_copy(zb_v, denom_sp.at[pl.ds(sid * STRIPE, STRIPE)])
    plsc.subcore_barrier()

    def _chunk(cc, carry):
        base = cc * K
        for v in range(K // 16):
            off = base + v * 16
            si = sidx_v[pl.ds(off, 16)]
            ri = ridxv_v[pl.ds(off, 16)]
            a_s = plsc.load_gather(scores_v, [si * 2])
            a_d = plsc.load_gather(scores_v, [ri * 2 + 1])
            t = a_s + a_d
            nl = jnp.maximum(t, t * NEG_SLOPE)
            ex_v[pl.ds(off, 16)] = jnp.exp(nl)
        pltpu.sync_copy(ex_v.at[pl.ds(base, K)],
                        denom_sp.at[ridx2_v.at[cc]], add=True)
        return carry
    lax.fori_loop(0, NCHUNK, _chunk, 0)

    pltpu.sync_copy(ex_v, ex_hbm.at[wid])
    plsc.subcore_barrier()
    pltpu.sync_copy(denom_sp.at[pl.ds(sid * STRIPE, STRIPE)], zb_v)
    pltpu.sync_copy(zb_v, denom_hbm.at[cid].at[pl.ds(sid * STRIPE, STRIPE)])

  return _sc_pass_a


@functools.cache
def _make_sc_pass_c():
  mesh = plsc.VectorSubcoreMesh(core_axis_name="c", subcore_axis_name="s")
  HK = K // 2

  @functools.partial(
      pl.kernel,
      mesh=mesh,
      out_type=[
          jax.ShapeDtypeStruct((NC, N_PAD, D), jnp.float32),
      ],
      compiler_params=pltpu.CompilerParams(needs_layout_passes=False),
      scratch_types=[
          pltpu.VMEM((2, K), jnp.int32),
          pltpu.VMEM((2, K), jnp.int32),
          pltpu.VMEM((2, K), jnp.float32),
          pltpu.VMEM((2, HK), jnp.int32),
          pltpu.VMEM((STRIPE,), jnp.float32),
          pltpu.VMEM((STRIPE,), jnp.float32),
          pltpu.VMEM((N_PAD,), jnp.float32),
          pltpu.VMEM((2, HK, D), jnp.float32),
          pltpu.VMEM((K,), jnp.float32),
          pltpu.VMEM((16, D), jnp.float32),
          pltpu.VMEM_SHARED((N_PAD, D), jnp.float32),
          pltpu.SemaphoreType.DMA,
          pltpu.SemaphoreType.DMA,
          pltpu.SemaphoreType.DMA,
      ],
  )
  def _sc_pass_c(x_hbm, sidx_hbm, ridx_hbm, ex_hbm, denom_hbm, agg_hbm,
                 sidxc_v, ridxc_v, exc_v, ridxs_v, d0_v, d1_v, inv_v,
                 rows_v, attn_v, zb_v, agg_sp, semlin, semga, semgb):
    cid = lax.axis_index("c")
    sid = lax.axis_index("s")
    wid = sid * NC + cid

    def _fire_lin(c, b):
        sl = pl.ds(c * K, K)
        pltpu.async_copy(sidx_hbm.at[wid].at[sl], sidxc_v.at[b], semlin)
        pltpu.async_copy(ridx_hbm.at[wid].at[sl], ridxc_v.at[b], semlin)
        pltpu.async_copy(ex_hbm.at[wid].at[sl], exc_v.at[b], semlin)

    def _drain_lin(b):
        sl = pl.ds(0, K)
        pltpu.make_async_copy(sidx_hbm.at[wid].at[sl], sidxc_v.at[b],
                              semlin).wait()
        pltpu.make_async_copy(ridx_hbm.at[wid].at[sl], ridxc_v.at[b],
                              semlin).wait()
        pltpu.make_async_copy(ex_hbm.at[wid].at[sl], exc_v.at[b],
                              semlin).wait()

    def _inv(i, carry):
        base = i * STRIPE
        pltpu.sync_copy(denom_hbm.at[0].at[pl.ds(base, STRIPE)], d0_v)
        pltpu.sync_copy(denom_hbm.at[1].at[pl.ds(base, STRIPE)], d1_v)

        def _inv16(j, carry2):
            sl = pl.ds(j * 16, 16)
            inv_v[pl.ds(base + j * 16, 16)] = 1.0 / jnp.maximum(
                d0_v[sl] + d1_v[sl], 1e-30)
            return carry2
        lax.fori_loop(0, STRIPE // 16, _inv16, 0)
        return carry
    lax.fori_loop(0, N_PAD // STRIPE, _inv, 0)

    for u in range(16):
        for v in range(D // 16):
            zb_v[u, pl.ds(v * 16, 16)] = jnp.zeros((16,), jnp.float32)

    def _zero(i, carry):
        pltpu.sync_copy(zb_v, agg_sp.at[pl.ds(sid * STRIPE + i * 16, 16)])
        return carry
    lax.fori_loop(0, STRIPE // 16, _zero, 0)
    plsc.subcore_barrier()

    _fire_lin(0, 0)

    def _half(b, h, g):
        g.wait()

        @plsc.parallel_loop(0, HK, unroll=4)
        def _rowscale(j):
            av = plsc.load_gather(attn_v, [jnp.zeros((16,), jnp.int32)
                                           + (j + h * HK)])
            for u in range(D // 16):
                sl = pl.ds(u * 16, 16)
                rows_v[h, j, sl] = rows_v[h, j, sl] * av
        pltpu.sync_copy(rows_v.at[h], agg_sp.at[ridxs_v.at[h]], add=True)

    def _step(c, b):
        _drain_lin(b)
        _fire_lin(jnp.minimum(c + 1, NCHUNK - 1), 1 - b)
        ga = pltpu.async_copy(x_hbm.at[sidxc_v.at[b].at[pl.ds(0, HK)]],
                              rows_v.at[0], semga)
        gb = pltpu.async_copy(x_hbm.at[sidxc_v.at[b].at[pl.ds(HK, HK)]],
                              rows_v.at[1], semgb)
        for h in range(2):
            for v in range(HK // 16):
                ridxs_v[h, pl.ds(v * 16, 16)] = (
                    ridxc_v[b, pl.ds(h * HK + v * 16, 16)])
        for v in range(K // 16):
            sl = pl.ds(v * 16, 16)
            ri = ridxc_v[b, sl]
            invv = plsc.load_gather(inv_v, [ri])
            attn_v[sl] = exc_v[b, sl] * invv
        _half(b, 0, ga)
        _half(b, 1, gb)

    def _pair(oo, carry):
        _step(oo * 2, 0)
        _step(oo * 2 + 1, 1)
        return carry
    lax.fori_loop(0, NCHUNK // 2, _pair, 0)
    _drain_lin(0)
    plsc.subcore_barrier()

    def _out(i, carry):
        row = sid * STRIPE + i * 16
        pltpu.sync_copy(agg_sp.at[pl.ds(row, 16)], zb_v)
        pltpu.sync_copy(zb_v, agg_hbm.at[cid].at[pl.ds(row, 16)])
        return carry
    lax.fori_loop(0, STRIPE // 16, _out, 0)

  return _sc_pass_c


def kernel(nodes, senders, receivers, enc_w, enc_b, attn0_w, attn0_b, core0_w,
           core0_b, attn1_w, attn1_b, core1_w, core1_b, dec_w, dec_b):
    nodes_p = jnp.pad(nodes, ((0, N_PAD - N), (0, 0)))
    s_idx, r_idx = receivers, senders
    s2 = jnp.pad(s_idx.reshape(NW, E // NW), ((0, 0), (0, EPT - E // NW)),
                 constant_values=0)
    r2 = jnp.pad(r_idx.reshape(NW, E // NW), ((0, 0), (0, EPT - E // NW)),
                 constant_values=N)
    r3 = r2.reshape(NW, NCHUNK, K)

    x, scores = _tc_encode(nodes_p, enc_w, enc_b, attn0_w, attn0_b)
    for hop, (aw, ab, cw, cb) in enumerate(
            ((attn0_w, attn0_b, core0_w, core0_b),
             (attn1_w, attn1_b, core1_w, core1_b))):
        ex, denom = _make_sc_pass_a()(scores.reshape(2 * N_PAD), s2, r2, r3)
        (agg2,) = _make_sc_pass_c()(x, s2, r2, ex, denom)
        if hop == 0:
            x, scores = _tc_update(agg2, cw, cb, attn1_w, attn1_b)
        else:
            out = _tc_decode(agg2, cw, cb, dec_w, dec_b)
    return out[:N]
